# baseline (speedup 1.0000x reference)
import functools

import jax
import jax.numpy as jnp
from jax import lax
from jax.experimental import pallas as pl
from jax.experimental.pallas import tpu as pltpu
from jax.experimental.pallas import tpu_sc as plsc

ALPHA = 0.8
_NC = 2
_NS = 16
_NW = _NC * _NS


def _safe_pow(x, p):
    safe = jnp.where(x > 0, x, 1.0)
    return jnp.where(x > 0, jnp.exp(p * jnp.log(safe)), 0.0)


def _compute_body(cls_ref, box_ref, gtt_ref, lab_ref, out_ref):
    sig = jax.nn.sigmoid(cls_ref[...])
    n_cls = cls_ref.shape[1]
    n_gt = lab_ref.shape[1]
    classes = lax.broadcasted_iota(jnp.int32, (n_cls, n_gt), 0)
    onehot = (classes == lab_ref[...]).astype(jnp.float32)
    cls_cost = jnp.dot(sig, onehot, preferred_element_type=jnp.float32)

    box = box_ref[...]
    px1, py1, px2, py2 = (box[:, 0:1], box[:, 1:2], box[:, 2:3], box[:, 3:4])
    gx1 = gtt_ref[0:1, :]
    gy1 = gtt_ref[1:2, :]
    gx2 = gtt_ref[2:3, :]
    gy2 = gtt_ref[3:4, :]
    area_p = (px2 - px1) * (py2 - py1)
    area_g = (gx2 - gx1) * (gy2 - gy1)
    iw = jnp.clip(jnp.minimum(px2, gx2) - jnp.maximum(px1, gx1), 0.0)
    ih = jnp.clip(jnp.minimum(py2, gy2) - jnp.maximum(py1, gy1), 0.0)
    inter = iw * ih
    union = area_p + area_g - inter
    iou = inter / jnp.maximum(union, 1e-6)

    ov = _safe_pow(cls_cost, 1.0 - ALPHA) * _safe_pow(iou, ALPHA)
    q = jnp.max(ov, axis=1, keepdims=True)
    out_ref[...] = jnp.where(q < 0.0, 0.0, q)


def _make_gather(n_pos, n_cls):
    b_per_w = n_pos // _NW
    mesh = plsc.VectorSubcoreMesh(core_axis_name="c", subcore_axis_name="s")

    @functools.partial(
        pl.kernel,
        out_type=[
            jax.ShapeDtypeStruct((n_pos, n_cls), jnp.float32),
            jax.ShapeDtypeStruct((n_pos, 16), jnp.float32),
        ],
        mesh=mesh,
        scratch_types=[
            pltpu.VMEM((b_per_w,), jnp.int32),
            pltpu.VMEM((b_per_w, n_cls), jnp.float32),
            pltpu.VMEM((b_per_w, 16), jnp.float32),
            pltpu.SemaphoreType.DMA,
            pltpu.SemaphoreType.DMA,
        ],
        compiler_params=pltpu.CompilerParams(use_tc_tiling_on_sc=False),
    )
    def gather_k(cls_hbm, box_hbm, idx_hbm, cls_out, box_out,
                 idx_v, cls_v, box_v, sem_c, sem_b):
        wid = lax.axis_index("s") * _NC + lax.axis_index("c")
        base = wid * b_per_w
        pltpu.sync_copy(idx_hbm.at[pl.ds(base, b_per_w)], idx_v)
        cp_c = pltpu.async_copy(cls_hbm.at[idx_v], cls_v, sem_c)
        cp_b = pltpu.async_copy(box_hbm.at[idx_v], box_v, sem_b)
        cp_c.wait()
        cp_b.wait()
        pltpu.sync_copy(cls_v, cls_out.at[pl.ds(base, b_per_w)])
        pltpu.sync_copy(box_v, box_out.at[pl.ds(base, b_per_w)])

    return gather_k


def _make_scatter(n_pos, n_pad):
    chunk = n_pad // _NW
    mesh = plsc.VectorSubcoreMesh(core_axis_name="c", subcore_axis_name="s")

    @functools.partial(
        pl.kernel,
        out_type=jax.ShapeDtypeStruct((n_pad,), jnp.float32),
        mesh=mesh,
        scratch_types=[
            pltpu.VMEM((chunk,), jnp.float32),
            pltpu.VMEM((n_pos,), jnp.int32),
            pltpu.VMEM((n_pos,), jnp.float32),
        ],
        compiler_params=pltpu.CompilerParams(
            use_tc_tiling_on_sc=False, needs_layout_passes=False),
    )
    def scatter_k(idx_hbm, q_hbm, out_hbm, chunk_v, idx_v, q_v):
        wid = lax.axis_index("s") * _NC + lax.axis_index("c")
        off = wid * chunk
        zeros16 = jnp.zeros((16,), jnp.float32)
        for j in range(chunk // 16):
            chunk_v[pl.ds(j * 16, 16)] = zeros16
        pltpu.sync_copy(idx_hbm, idx_v)
        pltpu.sync_copy(q_hbm, q_v)
        for j in range(n_pos // 16):
            iv = idx_v[pl.ds(j * 16, 16)]
            qv = q_v[pl.ds(j * 16, 16)]
            m = (iv >= off) & (iv < off + chunk)
            loc = jnp.where(m, iv - off, 0)
            plsc.store_scatter(chunk_v, [loc], qv, mask=m)
        pltpu.sync_copy(chunk_v, out_hbm.at[pl.ds(off, chunk)])

    return scatter_k


def kernel(bboxes, cls_scores, bbox_preds, gt_bboxes, bbox_levels,
           positive_inds, gt_labels):
    n = bboxes.shape[0]
    n_cls = cls_scores.shape[1]
    n_pos = positive_inds.shape[0]

    idx = positive_inds.astype(jnp.int32)
    box_pad = jnp.pad(bbox_preds.astype(jnp.float32), ((0, 0), (0, 12)))

    cls_g, box_g = _make_gather(n_pos, n_cls)(
        cls_scores.astype(jnp.float32), box_pad, idx)

    gt_t = gt_bboxes.astype(jnp.float32).T
    lab = gt_labels.astype(jnp.int32).reshape(1, -1)
    q = pl.pallas_call(
        _compute_body,
        out_shape=jax.ShapeDtypeStruct((n_pos, 1), jnp.float32),
    )(cls_g, box_g, gt_t, lab)

    per_w = 16 * _NW
    n_pad = ((n + per_w - 1) // per_w) * per_w
    out_pad = _make_scatter(n_pos, n_pad)(idx, q.reshape(-1))
    return out_pad[:n]
